# baseline (speedup 1.0000x reference)
import functools

import jax
import jax.numpy as jnp
from jax import lax
from jax.experimental import pallas as pl
from jax.experimental.pallas import tpu as pltpu
from jax.experimental.pallas import tpu_sc as plsc

BEAM_N = 32
VOCAB_N = 100000
EOS_TOK = 2
K = 32
LANES = 16
SEG = 400
VPS = SEG // LANES
NSEG = VOCAB_N // SEG
NSEG_PAD = 256
GRP = 16
NGRP = NSEG_PAD // GRP
NCHUNK = 5
CHUNK = VOCAB_N // NCHUNK
SEG_PER_CHUNK = CHUNK // SEG
NEG = -3.0e38
MASKVAL = -1.0e20
BIGI = 2**30


def _sc_body(probs_hbm, scores_hbm, prev_hbm, ovals_hbm, ocols_hbm,
             row_v, m1_v, m2_v, vals_v, cols_v, sc_v, pt_v, dsem):
    w = lax.axis_index("s") * 2 + lax.axis_index("c")
    rowcopy = pltpu.async_copy(probs_hbm.at[w], row_v, dsem.at[0])
    pltpu.sync_copy(scores_hbm, sc_v)
    pltpu.sync_copy(prev_hbm, pt_v)
    rowcopy.wait()

    def seg_body(s, carry):
        base = s * SEG
        acc = row_v[pl.ds(base, LANES)]
        for j in range(1, VPS):
            acc = jnp.maximum(acc, row_v[pl.ds(base + j * LANES, LANES)])
        m1_v[pl.ds(s * LANES, LANES)] = acc
        return carry

    lax.fori_loop(0, NSEG, seg_body, 0)

    neg_vec = jnp.full((LANES,), jnp.float32(NEG))
    for s in range(NSEG, NSEG_PAD):
        m1_v[pl.ds(s * LANES, LANES)] = neg_vec

    def grp_body(g, carry):
        gb = g * GRP
        acc = m1_v[pl.ds(gb * LANES, LANES)]
        for j in range(1, GRP):
            acc = jnp.maximum(acc, m1_v[pl.ds((gb + j) * LANES, LANES)])
        m2_v[pl.ds(g * LANES, LANES)] = acc
        return carry

    lax.fori_loop(0, NGRP, grp_body, 0)

    lane_iota = lax.iota(jnp.int32, LANES)

    lane0 = lane_iota == 0

    def ext_body(i, carry):
        v0, v1, c0, c1 = carry
        m3 = m2_v[pl.ds(0, LANES)]
        for g in range(1, NGRP):
            m3 = jnp.maximum(m3, m2_v[pl.ds(g * LANES, LANES)])
        m = jnp.max(m3)

        gsel = jnp.full((LANES,), BIGI, jnp.int32)
        for g in range(NGRP):
            gsel = jnp.minimum(gsel, jnp.where(
                m2_v[pl.ds(g * LANES, LANES)] == m, jnp.int32(g),
                jnp.int32(BIGI)))
        gstar = jnp.min(gsel)

        ssel = jnp.full((LANES,), BIGI, jnp.int32)
        gbase = gstar * GRP
        for j in range(GRP):
            ssel = jnp.minimum(
                ssel, jnp.where(m1_v[pl.ds((gbase + j) * LANES, LANES)] == m,
                                gbase + j, jnp.int32(BIGI)))
        sstar = jnp.min(ssel)

        sbase = sstar * SEG
        csel = jnp.full((LANES,), BIGI, jnp.int32)
        for j in range(VPS):
            off = sbase + j * LANES
            eq = row_v[pl.ds(off, LANES)] == m
            csel = jnp.minimum(csel, jnp.where(eq, off + lane_iota,
                                               jnp.int32(BIGI)))
        cstar = jnp.min(csel)

        sel0 = lane_iota == i
        sel1 = lane_iota == (i - LANES)
        v0 = jnp.where(sel0, m, v0)
        v1 = jnp.where(sel1, m, v1)
        c0 = jnp.where(sel0, cstar, c0)
        c1 = jnp.where(sel1, cstar, c1)
        plsc.store_scatter(
            row_v, [jnp.full((LANES,), 0, jnp.int32) + cstar],
            jnp.full((LANES,), jnp.float32(NEG)), mask=lane0)

        acc = row_v[pl.ds(sbase, LANES)]
        for j in range(1, VPS):
            acc = jnp.maximum(acc, row_v[pl.ds(sbase + j * LANES, LANES)])
        m1_v[pl.ds(sstar * LANES, LANES)] = acc

        acc2 = m1_v[pl.ds(gbase * LANES, LANES)]
        for j in range(1, GRP):
            acc2 = jnp.maximum(acc2, m1_v[pl.ds((gbase + j) * LANES, LANES)])
        m2_v[pl.ds(gstar * LANES, LANES)] = acc2
        return v0, v1, c0, c1

    zf = jnp.zeros((LANES,), jnp.float32)
    zi = jnp.zeros((LANES,), jnp.int32)
    v0, v1, c0, c1 = lax.fori_loop(0, K, ext_body, (zf, zf, zi, zi))

    widx = jnp.full((LANES,), 0, jnp.int32) + w
    score_w = plsc.load_gather(sc_v, [widx])
    is_eos = plsc.load_gather(pt_v, [widx]) == EOS_TOK
    for h, (v, c) in enumerate(((v0, c0), (v1, c1))):
        li = lane_iota + h * LANES
        vals_v[pl.ds(h * LANES, LANES)] = jnp.where(
            is_eos, jnp.float32(MASKVAL), v + score_w)
        cols_v[pl.ds(h * LANES, LANES)] = jnp.where(is_eos, li, c)

    pltpu.sync_copy(vals_v, ovals_hbm.at[0, pl.ds(w * K, K)])
    pltpu.sync_copy(cols_v, ocols_hbm.at[0, pl.ds(w * K, K)])


_sc_rows_topk = functools.partial(
    pl.kernel,
    mesh=plsc.VectorSubcoreMesh(core_axis_name="c", subcore_axis_name="s"),
    compiler_params=pltpu.CompilerParams(needs_layout_passes=False),
    out_type=[
        jax.ShapeDtypeStruct((1, BEAM_N * K), jnp.float32),
        jax.ShapeDtypeStruct((1, BEAM_N * K), jnp.int32),
    ],
    scratch_types=[
        pltpu.VMEM((VOCAB_N,), jnp.float32),
        pltpu.VMEM((NSEG_PAD * LANES,), jnp.float32),
        pltpu.VMEM((NGRP * LANES,), jnp.float32),
        pltpu.VMEM((K,), jnp.float32),
        pltpu.VMEM((K,), jnp.int32),
        pltpu.VMEM((BEAM_N,), jnp.float32),
        pltpu.VMEM((BEAM_N,), jnp.int32),
        pltpu.SemaphoreType.DMA((NCHUNK,)),
    ],
)(_sc_body)


NC = BEAM_N * K


def _merge_body(vr_ref, cr_ref, bs_ref, hy_ref, tk_ref):
    v1 = vr_ref[...]
    f1 = ((lax.broadcasted_iota(jnp.int32, (1, NC), 1) // K) * VOCAB_N
          + cr_ref[...])
    v = jnp.concatenate([v1[:, k * 128:(k + 1) * 128] for k in range(8)],
                        axis=0)
    fr = jnp.concatenate([f1[:, k * 128:(k + 1) * 128] for k in range(8)],
                         axis=0)
    big = jnp.int32(2**31 - 1)
    lane = lax.broadcasted_iota(jnp.int32, (1, K), 1)

    def allreduce(x, op):
        for sh in (1, 2, 4, 8, 16, 32, 64):
            x = op(x, pltpu.roll(x, sh, axis=1))
        for sh in (1, 2, 4):
            x = op(x, pltpu.roll(x, sh, axis=0))
        return x

    def body(i, carry):
        v, best, hyp, tok = carry
        m = allreduce(v, jnp.maximum)
        fmin = allreduce(jnp.where(v == m, fr, big), jnp.minimum)
        sel = lane == i
        m32 = lax.slice(m, (0, 0), (1, K))
        f32v = lax.slice(fmin, (0, 0), (1, K))
        best = jnp.where(sel, m32, best)
        hyp = jnp.where(sel, f32v // VOCAB_N, hyp)
        tok = jnp.where(sel, f32v % VOCAB_N, tok)
        v = jnp.where(fr == fmin, jnp.float32(NEG), v)
        return v, best, hyp, tok

    carry = (v,
             jnp.zeros((1, K), jnp.float32),
             jnp.zeros((1, K), jnp.int32),
             jnp.zeros((1, K), jnp.int32))
    for i in range(K):
        carry = body(i, carry)
    _, best, hyp, tok = carry
    bs_ref[...] = best
    hy_ref[...] = hyp
    tk_ref[...] = tok


def kernel(softmax_probs, scores, prev_tokens):
    cand_vals, cand_cols = _sc_rows_topk(softmax_probs, scores, prev_tokens)
    best, hyp, tok = pl.pallas_call(
        _merge_body,
        out_shape=[
            jax.ShapeDtypeStruct((1, K), jnp.float32),
            jax.ShapeDtypeStruct((1, K), jnp.int32),
            jax.ShapeDtypeStruct((1, K), jnp.int32),
        ],
    )(cand_vals, cand_cols)
    return best.reshape(K), hyp.reshape(K), tok.reshape(K)


# speedup vs baseline: 1.4376x; 1.4376x over previous
import functools

import jax
import jax.numpy as jnp
from jax import lax
from jax.experimental import pallas as pl
from jax.experimental.pallas import tpu as pltpu
from jax.experimental.pallas import tpu_sc as plsc

BEAM_N = 32
VOCAB_N = 100000
EOS_TOK = 2
K = 32
LANES = 16
SEG = 400
VPS = SEG // LANES
NSEG = VOCAB_N // SEG
NSEG_PAD = 256
GRP = 16
NGRP = NSEG_PAD // GRP
NCHUNK = 5
CHUNK = VOCAB_N // NCHUNK
SEG_PER_CHUNK = CHUNK // SEG
NEG = -3.0e38
MASKVAL = -1.0e20
BIGI = 2**30


def _sc_body(probs_hbm, scores_hbm, prev_hbm, ovals_hbm, ocols_hbm,
             row_v, m1_v, m2_v, vals_v, cols_v, sc_v, pt_v, dsem):
    w = lax.axis_index("s") * 2 + lax.axis_index("c")
    rowcopy = pltpu.async_copy(probs_hbm.at[w], row_v, dsem.at[0])
    pltpu.sync_copy(scores_hbm, sc_v)
    pltpu.sync_copy(prev_hbm, pt_v)
    rowcopy.wait()

    def seg_body(s, carry):
        base = s * SEG
        acc = row_v[pl.ds(base, LANES)]
        for j in range(1, VPS):
            acc = jnp.maximum(acc, row_v[pl.ds(base + j * LANES, LANES)])
        m1_v[pl.ds(s * LANES, LANES)] = acc
        return carry

    lax.fori_loop(0, NSEG, seg_body, 0)

    neg_vec = jnp.full((LANES,), jnp.float32(NEG))
    for s in range(NSEG, NSEG_PAD):
        m1_v[pl.ds(s * LANES, LANES)] = neg_vec

    def grp_body(g, carry):
        gb = g * GRP
        acc = m1_v[pl.ds(gb * LANES, LANES)]
        for j in range(1, GRP):
            acc = jnp.maximum(acc, m1_v[pl.ds((gb + j) * LANES, LANES)])
        m2_v[pl.ds(g * LANES, LANES)] = acc
        return carry

    lax.fori_loop(0, NGRP, grp_body, 0)

    lane_iota = lax.iota(jnp.int32, LANES)

    lane0 = lane_iota == 0

    def ext_body(i, carry):
        v0, v1, c0, c1 = carry
        m3 = m2_v[pl.ds(0, LANES)]
        for g in range(1, NGRP):
            m3 = jnp.maximum(m3, m2_v[pl.ds(g * LANES, LANES)])
        m = jnp.max(m3)

        gsel = jnp.full((LANES,), BIGI, jnp.int32)
        for g in range(NGRP):
            gsel = jnp.minimum(gsel, jnp.where(
                m2_v[pl.ds(g * LANES, LANES)] == m, jnp.int32(g),
                jnp.int32(BIGI)))
        gstar = jnp.min(gsel)

        ssel = jnp.full((LANES,), BIGI, jnp.int32)
        gbase = gstar * GRP
        for j in range(GRP):
            ssel = jnp.minimum(
                ssel, jnp.where(m1_v[pl.ds((gbase + j) * LANES, LANES)] == m,
                                gbase + j, jnp.int32(BIGI)))
        sstar = jnp.min(ssel)

        sbase = sstar * SEG
        csel = jnp.full((LANES,), BIGI, jnp.int32)
        for j in range(VPS):
            off = sbase + j * LANES
            eq = row_v[pl.ds(off, LANES)] == m
            csel = jnp.minimum(csel, jnp.where(eq, off + lane_iota,
                                               jnp.int32(BIGI)))
        cstar = jnp.min(csel)

        sel0 = lane_iota == i
        sel1 = lane_iota == (i - LANES)
        v0 = jnp.where(sel0, m, v0)
        v1 = jnp.where(sel1, m, v1)
        c0 = jnp.where(sel0, cstar, c0)
        c1 = jnp.where(sel1, cstar, c1)
        plsc.store_scatter(
            row_v, [jnp.full((LANES,), 0, jnp.int32) + cstar],
            jnp.full((LANES,), jnp.float32(NEG)), mask=lane0)

        acc = row_v[pl.ds(sbase, LANES)]
        for j in range(1, VPS):
            acc = jnp.maximum(acc, row_v[pl.ds(sbase + j * LANES, LANES)])
        m1_v[pl.ds(sstar * LANES, LANES)] = acc

        acc2 = m1_v[pl.ds(gbase * LANES, LANES)]
        for j in range(1, GRP):
            acc2 = jnp.maximum(acc2, m1_v[pl.ds((gbase + j) * LANES, LANES)])
        m2_v[pl.ds(gstar * LANES, LANES)] = acc2
        return v0, v1, c0, c1

    zf = jnp.zeros((LANES,), jnp.float32)
    zi = jnp.zeros((LANES,), jnp.int32)
    v0, v1, c0, c1 = lax.fori_loop(0, K, ext_body, (zf, zf, zi, zi))

    widx = jnp.full((LANES,), 0, jnp.int32) + w
    score_w = plsc.load_gather(sc_v, [widx])
    is_eos = plsc.load_gather(pt_v, [widx]) == EOS_TOK
    for h, (v, c) in enumerate(((v0, c0), (v1, c1))):
        li = lane_iota + h * LANES
        vals_v[pl.ds(h * LANES, LANES)] = jnp.where(
            is_eos, jnp.float32(MASKVAL), v + score_w)
        cols_v[pl.ds(h * LANES, LANES)] = jnp.where(is_eos, li, c)

    pltpu.sync_copy(vals_v, ovals_hbm.at[0, pl.ds(w * K, K)])
    pltpu.sync_copy(cols_v, ocols_hbm.at[0, pl.ds(w * K, K)])


_sc_rows_topk = functools.partial(
    pl.kernel,
    mesh=plsc.VectorSubcoreMesh(core_axis_name="c", subcore_axis_name="s"),
    compiler_params=pltpu.CompilerParams(needs_layout_passes=False),
    out_type=[
        jax.ShapeDtypeStruct((1, BEAM_N * K), jnp.float32),
        jax.ShapeDtypeStruct((1, BEAM_N * K), jnp.int32),
    ],
    scratch_types=[
        pltpu.VMEM((VOCAB_N,), jnp.float32),
        pltpu.VMEM((NSEG_PAD * LANES,), jnp.float32),
        pltpu.VMEM((NGRP * LANES,), jnp.float32),
        pltpu.VMEM((K,), jnp.float32),
        pltpu.VMEM((K,), jnp.int32),
        pltpu.VMEM((BEAM_N,), jnp.float32),
        pltpu.VMEM((BEAM_N,), jnp.int32),
        pltpu.SemaphoreType.DMA((NCHUNK,)),
    ],
)(_sc_body)


NC = BEAM_N * K


def _merge_body(vr_ref, cr_ref, bs_ref, hy_ref, tk_ref):
    v1 = vr_ref[...]
    f1 = ((lax.broadcasted_iota(jnp.int32, (1, NC), 1) // K) * VOCAB_N
          + cr_ref[...])
    v = jnp.concatenate([v1[:, k * 128:(k + 1) * 128] for k in range(8)],
                        axis=0)
    fr = jnp.concatenate([f1[:, k * 128:(k + 1) * 128] for k in range(8)],
                         axis=0)
    big = jnp.int32(2**31 - 1)
    lane = lax.broadcasted_iota(jnp.int32, (1, K), 1)

    def body(i, carry):
        v, best, hyp, tok = carry
        m = jnp.max(v)
        fmin = jnp.min(jnp.where(v == m, fr, big))
        sel = lane == i
        best = jnp.where(sel, m, best)
        hyp = jnp.where(sel, fmin // VOCAB_N, hyp)
        tok = jnp.where(sel, fmin % VOCAB_N, tok)
        v = jnp.where(fr == fmin, jnp.float32(NEG), v)
        return v, best, hyp, tok

    carry = (v,
             jnp.zeros((1, K), jnp.float32),
             jnp.zeros((1, K), jnp.int32),
             jnp.zeros((1, K), jnp.int32))
    for i in range(K):
        carry = body(i, carry)
    _, best, hyp, tok = carry
    bs_ref[...] = best
    hy_ref[...] = hyp
    tk_ref[...] = tok


def kernel(softmax_probs, scores, prev_tokens):
    cand_vals, cand_cols = _sc_rows_topk(softmax_probs, scores, prev_tokens)
    best, hyp, tok = pl.pallas_call(
        _merge_body,
        out_shape=[
            jax.ShapeDtypeStruct((1, K), jnp.float32),
            jax.ShapeDtypeStruct((1, K), jnp.int32),
            jax.ShapeDtypeStruct((1, K), jnp.int32),
        ],
    )(cand_vals, cand_cols)
    return best.reshape(K), hyp.reshape(K), tok.reshape(K)
